# Initial kernel scaffold; baseline (speedup 1.0000x reference)
import jax
import jax.numpy as jnp
from jax.experimental import pallas as pl


def kernel(x):
    raise NotImplementedError("write your pallas kernel here")


# speedup vs baseline: 2.6663x; 2.6663x over previous
import functools

import jax
import jax.numpy as jnp
from jax import lax
from jax.experimental import pallas as pl
from jax.experimental.pallas import tpu as pltpu
from jax.experimental.pallas import tpu_sc as plsc

_NC = 2
_NS = 16
_NW = _NC * _NS


def _ceil16(v):
    return ((v + 15) >> 4) << 4


@functools.cache
def _build(N: int):
    total = N * (N + 1) // 2
    corner_lo = N - 32
    corner_elems = 528
    corner_base = total - corner_elems

    bands = []
    P = N
    while P >= 64:
        bands.append((P, N - P, N - P // 2))
        P //= 2

    mesh = plsc.VectorSubcoreMesh(core_axis_name="c", subcore_axis_name="s")

    @functools.partial(
        pl.kernel,
        mesh=mesh,
        out_type=jax.ShapeDtypeStruct((total,), jnp.float32),
        scratch_types=[
            pltpu.VMEM((N + 64, ), jnp.float32),
            pltpu.VMEM((N + 16, ), jnp.float32),
            pltpu.VMEM((32, ), jnp.float32),
            pltpu.VMEM((16, ), jnp.float32),
            pltpu.VMEM((2112, ), jnp.float32),
            pltpu.VMEM((corner_elems, ), jnp.float32),
        ],
    )
    def triu_kernel(x_hbm, out_hbm, buf, buf2, buf3, st, csrc, cout):
        c = lax.axis_index("c")
        s = lax.axis_index("s")
        w = c * _NS + s

        lanes = lax.iota(jnp.int32, 16)

        def general_row(r, P, C, SZ):
            ln = N - r
            o_r = r * N - ((r * (r - 1)) >> 1)
            o_r1 = o_r + ln
            t_r = (r * (r + 1)) >> 1
            a_beg = _ceil16(o_r)
            b_beg = a_beg - 16
            gs = ((b_beg + t_r) >> 3) << 3
            u = b_beg + t_r - gs
            pltpu.sync_copy(x_hbm.at[pl.ds(pl.multiple_of(gs, 8), SZ)],
                            buf.at[pl.ds(0, SZ)])
            w16 = ((o_r1 >> 4) << 4) - a_beg
            nv = w16 >> 4

            def shift_body(k, carry):
                v = buf[pl.ds(16 + u + 16 * k, 16)]
                buf2[pl.ds(pl.multiple_of(16 * k, 8), 16)] = v
                return carry

            lax.fori_loop(jnp.int32(0), nv, shift_body, jnp.int32(0))

            t_p = t_r - r
            gs2 = ((b_beg + t_p) >> 3) << 3
            u2 = b_beg + t_p - gs2
            pltpu.sync_copy(x_hbm.at[pl.ds(pl.multiple_of(gs2, 8), 32)], buf3)
            va = buf3[pl.ds(u2, 16)]
            vb = buf[pl.ds(u, 16)]
            st[...] = jnp.where(lanes + b_beg < o_r, va, vb)
            pltpu.sync_copy(st, out_hbm.at[pl.ds(pl.multiple_of(b_beg, 8), 16)])

            nfull = w16 // C

            def sc_body(j, carry):
                off = j * C
                pltpu.sync_copy(
                    buf2.at[pl.ds(pl.multiple_of(off, 8), C)],
                    out_hbm.at[pl.ds(pl.multiple_of(a_beg + off, 8), C)])
                return carry

            lax.fori_loop(jnp.int32(0), nfull, sc_body, jnp.int32(0))
            toff = w16 - C
            pltpu.sync_copy(
                buf2.at[pl.ds(pl.multiple_of(toff, 8), C)],
                out_hbm.at[pl.ds(pl.multiple_of(a_beg + toff, 8), C)])

        for bi, (P, lo, hi) in enumerate(bands):
            C = max(16, P // 4)
            SZ = P + 64

            def band_body(i, carry, P=P, C=C, SZ=SZ, first=(bi == 0)):
                r = w + _NW * i
                if first:
                    @pl.when(r > 0)
                    def _():
                        general_row(r, P, C, SZ)

                    @pl.when(r == 0)
                    def _():
                        CH = min(2048, N)
                        for k in range(N // CH):
                            pltpu.sync_copy(x_hbm.at[pl.ds(k * CH, CH)],
                                            buf.at[pl.ds(0, CH)])
                            pltpu.sync_copy(buf.at[pl.ds(0, CH)],
                                            out_hbm.at[pl.ds(k * CH, CH)])
                else:
                    general_row(r, P, C, SZ)
                return carry

            lax.fori_loop(jnp.int32(lo // _NW), jnp.int32(hi // _NW),
                          band_body, jnp.int32(0))

        @pl.when(w == 1)
        def _():
            offs = []
            for j in range(32):
                r = corner_lo + j
                src0 = r * (N + 1)
                gs_j = (src0 >> 3) << 3
                u_j = src0 - gs_j
                size_j = min(56, N * N - gs_j)
                offs.append((r, u_j))
                pltpu.sync_copy(x_hbm.at[pl.ds(gs_j, size_j)],
                                csrc.at[pl.ds(64 * j + 16, size_j)])
            o_c = [((corner_lo + j) * N - ((corner_lo + j) * (corner_lo + j - 1)) // 2)
                   for j in range(33)]
            for t in range(33):
                base = corner_base + 16 * t
                vec = None
                for j in range(32):
                    if o_c[j + 1] <= base or o_c[j] >= base + 16:
                        continue
                    start = 64 * j + 16 + offs[j][1] + (base - o_c[j])
                    a = csrc[pl.ds(start, 16)]
                    if vec is None:
                        vec = a
                    else:
                        vec = jnp.where(lanes + base >= o_c[j], a, vec)
                cout[pl.ds(16 * t, 16)] = vec
            pltpu.sync_copy(cout, out_hbm.at[pl.ds(corner_base, corner_elems)])

    return triu_kernel


def kernel(x):
    n = x.shape[-1]
    flat = x.reshape(-1)
    return _build(n)(flat)


# speedup vs baseline: 4.8452x; 1.8172x over previous
import functools

import jax
import jax.numpy as jnp
from jax import lax
from jax.experimental import pallas as pl
from jax.experimental.pallas import tpu as pltpu
from jax.experimental.pallas import tpu_sc as plsc

_NC = 2
_NS = 16
_NW = _NC * _NS


def _ceil16(v):
    return ((v + 15) >> 4) << 4


@functools.cache
def _build(N: int):
    total = N * (N + 1) // 2
    corner_lo = N - 32
    corner_elems = 528
    corner_base = total - corner_elems

    bands = []
    P = N
    while P >= 64:
        bands.append((P, N - P, N - P // 2))
        P //= 2

    mesh = plsc.VectorSubcoreMesh(core_axis_name="c", subcore_axis_name="s")

    @functools.partial(
        pl.kernel,
        mesh=mesh,
        out_type=jax.ShapeDtypeStruct((total,), jnp.float32),
        scratch_types=[
            pltpu.VMEM((N + 64, ), jnp.float32),
            pltpu.VMEM((N + 64, ), jnp.float32),
            pltpu.VMEM((N + 16, ), jnp.float32),
            pltpu.VMEM((N + 16, ), jnp.float32),
            pltpu.VMEM((32, ), jnp.float32),
            pltpu.VMEM((32, ), jnp.float32),
            pltpu.VMEM((16, ), jnp.float32),
            pltpu.VMEM((16, ), jnp.float32),
            pltpu.VMEM((2112, ), jnp.float32),
            pltpu.VMEM((corner_elems, ), jnp.float32),
            pltpu.SemaphoreType.DMA,
            pltpu.SemaphoreType.DMA,
            pltpu.SemaphoreType.DMA,
            pltpu.SemaphoreType.DMA,
        ],
    )
    def triu_kernel(x_hbm, out_hbm, bufA, bufB, buf2A, buf2B, buf3A, buf3B,
                    stA, stB, csrc, cout, gsemA, gsemB, ssemA, ssemB):
        c = lax.axis_index("c")
        s = lax.axis_index("s")
        w = c * _NS + s

        lanes = lax.iota(jnp.int32, 16)
        slotA = (bufA, buf2A, buf3A, stA, gsemA, ssemA)
        slotB = (bufB, buf2B, buf3B, stB, gsemB, ssemB)

        def row_geom(r):
            ln = N - r
            o_r = r * N - ((r * (r - 1)) >> 1)
            t_r = (r * (r + 1)) >> 1
            a_beg = _ceil16(o_r)
            b_beg = a_beg - 16
            gs = ((b_beg + t_r) >> 3) << 3
            u = b_beg + t_r - gs
            t_p = t_r - r
            gs2 = ((b_beg + t_p) >> 3) << 3
            u2 = b_beg + t_p - gs2
            w16 = (((o_r + ln) >> 4) << 4) - a_beg
            return ln, o_r, a_beg, b_beg, gs, u, gs2, u2, w16

        def gather_descs(r, SZ, slot):
            buf, _, buf3, _, gsem, _ = slot
            _, _, _, _, gs, _, gs2, _, _ = row_geom(r)
            d1 = pltpu.make_async_copy(
                x_hbm.at[pl.ds(pl.multiple_of(gs, 8), SZ)],
                buf.at[pl.ds(0, SZ)], gsem)
            d2 = pltpu.make_async_copy(
                x_hbm.at[pl.ds(pl.multiple_of(gs2, 8), 32)], buf3, gsem)
            return d1, d2

        def row0_gather_descs(slot):
            buf, _, buf3, _, gsem, _ = slot
            d1 = pltpu.make_async_copy(x_hbm.at[pl.ds(0, N)],
                                       buf.at[pl.ds(16, N)], gsem)
            d2 = pltpu.make_async_copy(x_hbm.at[pl.ds(0, 32)], buf3, gsem)
            return d1, d2

        def fire_row(r, SZ, slot, first_band):
            if first_band:
                @pl.when(r > 0)
                def _():
                    for d in gather_descs(r, SZ, slot):
                        d.start()

                @pl.when(r == 0)
                def _():
                    for d in row0_gather_descs(slot):
                        d.start()
            else:
                for d in gather_descs(r, SZ, slot):
                    d.start()

        def scatter_descs(r, C, slot):
            _, buf2, _, st, _, ssem = slot
            _, _, a_beg, b_beg, _, _, _, _, w16 = row_geom(r)
            d_st = pltpu.make_async_copy(
                st, out_hbm.at[pl.ds(pl.multiple_of(b_beg, 8), 16)], ssem)
            toff = w16 - C
            d_tail = pltpu.make_async_copy(
                buf2.at[pl.ds(pl.multiple_of(toff, 8), C)],
                out_hbm.at[pl.ds(pl.multiple_of(a_beg + toff, 8), C)], ssem)
            return d_st, d_tail

        def chunk_desc(r, C, j, slot):
            _, buf2, _, _, _, ssem = slot
            _, _, a_beg, _, _, _, _, _, _ = row_geom(r)
            off = j * C
            return pltpu.make_async_copy(
                buf2.at[pl.ds(pl.multiple_of(off, 8), C)],
                out_hbm.at[pl.ds(pl.multiple_of(a_beg + off, 8), C)], ssem)

        def row0_scatter_desc(slot):
            _, buf2, _, _, _, ssem = slot
            return pltpu.make_async_copy(buf2.at[pl.ds(0, N)],
                                         out_hbm.at[pl.ds(0, N)], ssem)

        def nfull_of(r, C):
            _, _, _, _, _, _, _, _, w16 = row_geom(r)
            return w16 // C

        def fire_scatters(r, C, slot, first_band):
            def general():
                d_st, d_tail = scatter_descs(r, C, slot)
                d_st.start()
                d_tail.start()

                def body(j, carry):
                    chunk_desc(r, C, j, slot).start()
                    return carry

                lax.fori_loop(jnp.int32(0), nfull_of(r, C), body, jnp.int32(0))

            if first_band:
                @pl.when(r > 0)
                def _():
                    general()

                @pl.when(r == 0)
                def _():
                    row0_scatter_desc(slot).start()
            else:
                general()

        def drain_scatters(r, C, slot, first_band):
            def general():
                d_st, d_tail = scatter_descs(r, C, slot)
                d_st.wait()
                d_tail.wait()

                def body(j, carry):
                    chunk_desc(r, C, j, slot).wait()
                    return carry

                lax.fori_loop(jnp.int32(0), nfull_of(r, C), body, jnp.int32(0))

            if first_band:
                @pl.when(r > 0)
                def _():
                    general()

                @pl.when(r == 0)
                def _():
                    row0_scatter_desc(slot).wait()
            else:
                general()

        def consume_row(r, P, C, SZ, slot, first_band):
            buf, buf2, buf3, st, gsem, ssem = slot
            ln, o_r, a_beg, b_beg, gs, u, gs2, u2, w16 = row_geom(r)

            if first_band:
                @pl.when(r > 0)
                def _():
                    for d in gather_descs(r, SZ, slot):
                        d.wait()

                @pl.when(r == 0)
                def _():
                    for d in row0_gather_descs(slot):
                        d.wait()
                u_eff = jnp.where(r == 0, jnp.int32(0), u)
                a_eff = jnp.where(r == 0, jnp.int32(0), a_beg)
                w16_eff = (((o_r + ln) >> 4) << 4) - a_eff
            else:
                for d in gather_descs(r, SZ, slot):
                    d.wait()
                u_eff, w16_eff = u, w16

            def shift_body(k, carry):
                v = buf[pl.ds(16 + u_eff + 16 * k, 16)]
                buf2[pl.ds(pl.multiple_of(16 * k, 8), 16)] = v
                return carry

            lax.fori_loop(jnp.int32(0), w16_eff >> 4, shift_body, jnp.int32(0))

            if first_band:
                @pl.when(r > 0)
                def _():
                    va = buf3[pl.ds(u2, 16)]
                    vb = buf[pl.ds(u, 16)]
                    st[...] = jnp.where(lanes + b_beg < o_r, va, vb)
            else:
                va = buf3[pl.ds(u2, 16)]
                vb = buf[pl.ds(u, 16)]
                st[...] = jnp.where(lanes + b_beg < o_r, va, vb)

            fire_scatters(r, C, slot, first_band)

        for bi, (P, lo, hi) in enumerate(bands):
            C = max(16, P // 4)
            SZ = P + 64
            fb = (bi == 0)
            i_lo, i_hi = lo // _NW, hi // _NW
            npairs = (i_hi - i_lo) // 2

            fire_row(w + _NW * i_lo, SZ, slotA, fb)

            if npairs > 0:
                fire_row(w + _NW * (i_lo + 1), SZ, slotB, False)

                def pair_body(t, carry, P=P, C=C, SZ=SZ, fb=fb, i_lo=i_lo,
                              npairs=npairs):
                    rA = w + _NW * (i_lo + 2 * t)
                    rB = rA + _NW

                    @pl.when(t > 0)
                    def _():
                        drain_scatters(rA - 2 * _NW, C, slotA, fb)

                    consume_row(rA, P, C, SZ, slotA, fb)

                    @pl.when(t + 1 < npairs)
                    def _():
                        fire_row(rA + 2 * _NW, SZ, slotA, False)

                    @pl.when(t > 0)
                    def _():
                        drain_scatters(rB - 2 * _NW, C, slotB, False)

                    consume_row(rB, P, C, SZ, slotB, False)

                    @pl.when(t + 1 < npairs)
                    def _():
                        fire_row(rB + 2 * _NW, SZ, slotB, False)
                    return carry

                lax.fori_loop(jnp.int32(0), jnp.int32(npairs), pair_body,
                              jnp.int32(0))
                r_lastA = w + _NW * (i_lo + 2 * (npairs - 1))
                drain_scatters(r_lastA, C, slotA, fb)
                drain_scatters(r_lastA + _NW, C, slotB, False)
            else:
                r_only = w + _NW * i_lo
                consume_row(r_only, P, C, SZ, slotA, fb)
                drain_scatters(r_only, C, slotA, fb)

        @pl.when(w == 1)
        def _():
            cdescs = []
            for j in range(32):
                r = corner_lo + j
                src0 = r * (N + 1)
                gs_j = (src0 >> 3) << 3
                u_j = src0 - gs_j
                size_j = min(56, N * N - gs_j)
                d = pltpu.make_async_copy(
                    x_hbm.at[pl.ds(gs_j, size_j)],
                    csrc.at[pl.ds(64 * j + 16, size_j)], gsemA)
                d.start()
                cdescs.append((d, u_j))
            for d, _ in cdescs:
                d.wait()
            o_c = [((corner_lo + j) * N
                    - ((corner_lo + j) * (corner_lo + j - 1)) // 2)
                   for j in range(33)]
            for t in range(33):
                base = corner_base + 16 * t
                vec = None
                for j in range(32):
                    if o_c[j + 1] <= base or o_c[j] >= base + 16:
                        continue
                    start = 64 * j + 16 + cdescs[j][1] + (base - o_c[j])
                    a = csrc[pl.ds(start, 16)]
                    if vec is None:
                        vec = a
                    else:
                        vec = jnp.where(lanes + base >= o_c[j], a, vec)
                cout[pl.ds(16 * t, 16)] = vec
            pltpu.sync_copy(cout, out_hbm.at[pl.ds(corner_base, corner_elems)])

    return triu_kernel


def kernel(x):
    n = x.shape[-1]
    flat = x.reshape(-1)
    return _build(n)(flat)


# speedup vs baseline: 4.8508x; 1.0012x over previous
import functools

import jax
import jax.numpy as jnp
from jax import lax
from jax.experimental import pallas as pl
from jax.experimental.pallas import tpu as pltpu
from jax.experimental.pallas import tpu_sc as plsc

_NC = 2
_NS = 16
_NW = _NC * _NS


def _ceil16(v):
    return ((v + 15) >> 4) << 4


@functools.cache
def _build(N: int):
    total = N * (N + 1) // 2
    corner_lo = N - 32
    corner_elems = 528
    corner_base = total - corner_elems

    bands = []
    P = N
    while P >= 64:
        bands.append((P, N - P, N - P // 2))
        P //= 2

    mesh = plsc.VectorSubcoreMesh(core_axis_name="c", subcore_axis_name="s")

    @functools.partial(
        pl.kernel,
        mesh=mesh,
        out_type=jax.ShapeDtypeStruct((total,), jnp.float32),
        scratch_types=[
            pltpu.VMEM((N + 64, ), jnp.float32),
            pltpu.VMEM((N + 64, ), jnp.float32),
            pltpu.VMEM((N + 16, ), jnp.float32),
            pltpu.VMEM((N + 16, ), jnp.float32),
            pltpu.VMEM((32, ), jnp.float32),
            pltpu.VMEM((32, ), jnp.float32),
            pltpu.VMEM((16, ), jnp.float32),
            pltpu.VMEM((16, ), jnp.float32),
            pltpu.VMEM((2112, ), jnp.float32),
            pltpu.VMEM((corner_elems, ), jnp.float32),
            pltpu.SemaphoreType.DMA,
            pltpu.SemaphoreType.DMA,
            pltpu.SemaphoreType.DMA,
            pltpu.SemaphoreType.DMA,
        ],
    )
    def triu_kernel(x_hbm, out_hbm, bufA, bufB, buf2A, buf2B, buf3A, buf3B,
                    stA, stB, csrc, cout, gsemA, gsemB, ssemA, ssemB):
        c = lax.axis_index("c")
        s = lax.axis_index("s")
        w = c * _NS + s

        lanes = lax.iota(jnp.int32, 16)
        slotA = (bufA, buf2A, buf3A, stA, gsemA, ssemA)
        slotB = (bufB, buf2B, buf3B, stB, gsemB, ssemB)

        def row_geom(r):
            ln = N - r
            o_r = r * N - ((r * (r - 1)) >> 1)
            t_r = (r * (r + 1)) >> 1
            a_beg = _ceil16(o_r)
            b_beg = a_beg - 16
            gs = ((b_beg + t_r) >> 3) << 3
            u = b_beg + t_r - gs
            t_p = t_r - r
            gs2 = ((b_beg + t_p) >> 3) << 3
            u2 = b_beg + t_p - gs2
            w16 = (((o_r + ln) >> 4) << 4) - a_beg
            return ln, o_r, a_beg, b_beg, gs, u, gs2, u2, w16

        def gather_descs(r, SZ, slot):
            buf, _, buf3, _, gsem, _ = slot
            _, _, _, _, gs, _, gs2, _, _ = row_geom(r)
            d1 = pltpu.make_async_copy(
                x_hbm.at[pl.ds(pl.multiple_of(gs, 8), SZ)],
                buf.at[pl.ds(0, SZ)], gsem)
            d2 = pltpu.make_async_copy(
                x_hbm.at[pl.ds(pl.multiple_of(gs2, 8), 32)], buf3, gsem)
            return d1, d2

        def row0_gather_descs(slot):
            buf, _, buf3, _, gsem, _ = slot
            d1 = pltpu.make_async_copy(x_hbm.at[pl.ds(0, N)],
                                       buf.at[pl.ds(16, N)], gsem)
            d2 = pltpu.make_async_copy(x_hbm.at[pl.ds(0, 32)], buf3, gsem)
            return d1, d2

        def fire_row(r, SZ, slot, first_band):
            if first_band:
                @pl.when(r > 0)
                def _():
                    for d in gather_descs(r, SZ, slot):
                        d.start()

                @pl.when(r == 0)
                def _():
                    for d in row0_gather_descs(slot):
                        d.start()
            else:
                for d in gather_descs(r, SZ, slot):
                    d.start()

        def scatter_descs(r, C, slot):
            _, buf2, _, st, _, ssem = slot
            _, _, a_beg, b_beg, _, _, _, _, w16 = row_geom(r)
            d_st = pltpu.make_async_copy(
                st, out_hbm.at[pl.ds(pl.multiple_of(b_beg, 8), 16)], ssem)
            toff = w16 - C
            d_tail = pltpu.make_async_copy(
                buf2.at[pl.ds(pl.multiple_of(toff, 8), C)],
                out_hbm.at[pl.ds(pl.multiple_of(a_beg + toff, 8), C)], ssem)
            return d_st, d_tail

        def chunk_desc(r, C, j, slot):
            _, buf2, _, _, _, ssem = slot
            _, _, a_beg, _, _, _, _, _, _ = row_geom(r)
            off = j * C
            return pltpu.make_async_copy(
                buf2.at[pl.ds(pl.multiple_of(off, 8), C)],
                out_hbm.at[pl.ds(pl.multiple_of(a_beg + off, 8), C)], ssem)

        def row0_scatter_desc(slot):
            _, buf2, _, _, _, ssem = slot
            return pltpu.make_async_copy(buf2.at[pl.ds(0, N)],
                                         out_hbm.at[pl.ds(0, N)], ssem)

        def nfull_of(r, C):
            _, _, _, _, _, _, _, _, w16 = row_geom(r)
            return w16 // C

        def scatter_descs_direct(r, C, slot):
            buf, _, _, st, _, ssem = slot
            _, _, a_beg, b_beg, _, _, _, _, w16 = row_geom(r)
            d_st = pltpu.make_async_copy(
                st, out_hbm.at[pl.ds(pl.multiple_of(b_beg, 8), 16)], ssem)
            toff = w16 - C
            d_tail = pltpu.make_async_copy(
                buf.at[pl.ds(pl.multiple_of(16 + toff, 8), C)],
                out_hbm.at[pl.ds(pl.multiple_of(a_beg + toff, 8), C)], ssem)
            return d_st, d_tail

        def chunk_desc_direct(r, C, j, slot):
            buf, _, _, _, _, ssem = slot
            _, _, a_beg, _, _, _, _, _, _ = row_geom(r)
            off = j * C
            return pltpu.make_async_copy(
                buf.at[pl.ds(pl.multiple_of(16 + off, 8), C)],
                out_hbm.at[pl.ds(pl.multiple_of(a_beg + off, 8), C)], ssem)

        def fire_scatters(r, C, slot, first_band, u):
            del u
            def general():
                d_st, d_tail = scatter_descs(r, C, slot)
                d_st.start()
                d_tail.start()

                def body(j, carry):
                    chunk_desc(r, C, j, slot).start()
                    return carry

                lax.fori_loop(jnp.int32(0), nfull_of(r, C), body,
                              jnp.int32(0))

            if first_band:
                @pl.when(r > 0)
                def _():
                    general()

                @pl.when(r == 0)
                def _():
                    row0_scatter_desc(slot).start()
            else:
                general()

        def drain_scatters(r, C, slot, first_band):
            def general():
                d_st, d_tail = scatter_descs(r, C, slot)
                d_st.wait()
                d_tail.wait()

                def body(j, carry):
                    chunk_desc(r, C, j, slot).wait()
                    return carry

                lax.fori_loop(jnp.int32(0), nfull_of(r, C), body, jnp.int32(0))

            if first_band:
                @pl.when(r > 0)
                def _():
                    general()

                @pl.when(r == 0)
                def _():
                    row0_scatter_desc(slot).wait()
            else:
                general()

        def consume_row(r, P, C, SZ, slot, first_band):
            buf, buf2, buf3, st, gsem, ssem = slot
            ln, o_r, a_beg, b_beg, gs, u, gs2, u2, w16 = row_geom(r)

            if first_band:
                @pl.when(r > 0)
                def _():
                    for d in gather_descs(r, SZ, slot):
                        d.wait()

                @pl.when(r == 0)
                def _():
                    for d in row0_gather_descs(slot):
                        d.wait()
                u_eff = jnp.where(r == 0, jnp.int32(0), u)
                a_eff = jnp.where(r == 0, jnp.int32(0), a_beg)
                w16_eff = (((o_r + ln) >> 4) << 4) - a_eff
            else:
                for d in gather_descs(r, SZ, slot):
                    d.wait()
                u_eff, w16_eff = u, w16

            def shift_body(k, carry):
                v = buf[pl.ds(16 + u_eff + 16 * k, 16)]
                buf2[pl.ds(pl.multiple_of(16 * k, 8), 16)] = v
                return carry

            lax.fori_loop(jnp.int32(0), w16_eff >> 4, shift_body, jnp.int32(0))

            if first_band:
                @pl.when(r > 0)
                def _():
                    va = buf3[pl.ds(u2, 16)]
                    vb = buf[pl.ds(u, 16)]
                    st[...] = jnp.where(lanes + b_beg < o_r, va, vb)
            else:
                va = buf3[pl.ds(u2, 16)]
                vb = buf[pl.ds(u, 16)]
                st[...] = jnp.where(lanes + b_beg < o_r, va, vb)

            fire_scatters(r, C, slot, first_band, u_eff)

        for bi, (P, lo, hi) in enumerate(bands):
            C = 3 * P // 8 if P >= 256 else max(16, P // 4)
            SZ = P + 64
            fb = (bi == 0)
            i_lo, i_hi = lo // _NW, hi // _NW
            npairs = (i_hi - i_lo) // 2

            fire_row(w + _NW * i_lo, SZ, slotA, fb)

            if npairs > 0:
                fire_row(w + _NW * (i_lo + 1), SZ, slotB, False)

                def pair_body(t, carry, P=P, C=C, SZ=SZ, fb=fb, i_lo=i_lo,
                              npairs=npairs):
                    rA = w + _NW * (i_lo + 2 * t)
                    rB = rA + _NW

                    @pl.when(t > 0)
                    def _():
                        drain_scatters(rA - 2 * _NW, C, slotA, fb)

                    consume_row(rA, P, C, SZ, slotA, fb)

                    @pl.when(t + 1 < npairs)
                    def _():
                        fire_row(rA + 2 * _NW, SZ, slotA, False)

                    @pl.when(t > 0)
                    def _():
                        drain_scatters(rB - 2 * _NW, C, slotB, False)

                    consume_row(rB, P, C, SZ, slotB, False)

                    @pl.when(t + 1 < npairs)
                    def _():
                        fire_row(rB + 2 * _NW, SZ, slotB, False)
                    return carry

                lax.fori_loop(jnp.int32(0), jnp.int32(npairs), pair_body,
                              jnp.int32(0))
                r_lastA = w + _NW * (i_lo + 2 * (npairs - 1))
                drain_scatters(r_lastA, C, slotA, fb)
                drain_scatters(r_lastA + _NW, C, slotB, False)
            else:
                r_only = w + _NW * i_lo
                consume_row(r_only, P, C, SZ, slotA, fb)
                drain_scatters(r_only, C, slotA, fb)

        @pl.when(w == 1)
        def _():
            cdescs = []
            for j in range(32):
                r = corner_lo + j
                src0 = r * (N + 1)
                gs_j = (src0 >> 3) << 3
                u_j = src0 - gs_j
                size_j = min(56, N * N - gs_j)
                d = pltpu.make_async_copy(
                    x_hbm.at[pl.ds(gs_j, size_j)],
                    csrc.at[pl.ds(64 * j + 16, size_j)], gsemA)
                d.start()
                cdescs.append((d, u_j))
            for d, _ in cdescs:
                d.wait()
            o_c = [((corner_lo + j) * N
                    - ((corner_lo + j) * (corner_lo + j - 1)) // 2)
                   for j in range(33)]
            for t in range(33):
                base = corner_base + 16 * t
                vec = None
                for j in range(32):
                    if o_c[j + 1] <= base or o_c[j] >= base + 16:
                        continue
                    start = 64 * j + 16 + cdescs[j][1] + (base - o_c[j])
                    a = csrc[pl.ds(start, 16)]
                    if vec is None:
                        vec = a
                    else:
                        vec = jnp.where(lanes + base >= o_c[j], a, vec)
                cout[pl.ds(16 * t, 16)] = vec
            pltpu.sync_copy(cout, out_hbm.at[pl.ds(corner_base, corner_elems)])

    return triu_kernel


def kernel(x):
    n = x.shape[-1]
    flat = x.reshape(-1)
    return _build(n)(flat)


# speedup vs baseline: 6.2153x; 1.2813x over previous
import functools

import jax
import jax.numpy as jnp
from jax import lax
from jax.experimental import pallas as pl
from jax.experimental.pallas import tpu as pltpu
from jax.experimental.pallas import tpu_sc as plsc

_NC = 2
_NS = 16
_NW = _NC * _NS


def _ceil16(v):
    return ((v + 15) >> 4) << 4


@functools.cache
def _build(N: int):
    total = N * (N + 1) // 2
    corner_lo = N - 32
    corner_elems = 528
    corner_base = total - corner_elems

    bands = []
    P = N
    while P >= 64:
        bands.append((P, N - P, N - P // 2))
        P //= 2

    mesh = plsc.VectorSubcoreMesh(core_axis_name="c", subcore_axis_name="s")

    @functools.partial(
        pl.kernel,
        mesh=mesh,
        out_type=jax.ShapeDtypeStruct((total,), jnp.float32),
        scratch_types=[
            pltpu.VMEM((N + 64, ), jnp.float32),
            pltpu.VMEM((N + 64, ), jnp.float32),
            pltpu.VMEM((N + 16, ), jnp.float32),
            pltpu.VMEM((N + 16, ), jnp.float32),
            pltpu.VMEM((32, ), jnp.float32),
            pltpu.VMEM((32, ), jnp.float32),
            pltpu.VMEM((16, ), jnp.float32),
            pltpu.VMEM((16, ), jnp.float32),
            pltpu.VMEM((2112, ), jnp.float32),
            pltpu.VMEM((corner_elems, ), jnp.float32),
            pltpu.SemaphoreType.DMA,
            pltpu.SemaphoreType.DMA,
            pltpu.SemaphoreType.DMA,
            pltpu.SemaphoreType.DMA,
        ],
    )
    def triu_kernel(x_hbm, out_hbm, bufA, bufB, buf2A, buf2B, buf3A, buf3B,
                    stA, stB, csrc, cout, gsemA, gsemB, ssemA, ssemB):
        c = lax.axis_index("c")
        s = lax.axis_index("s")
        w = c * _NS + s

        lanes = lax.iota(jnp.int32, 16)
        slotA = (bufA, buf2A, buf3A, stA, gsemA, ssemA)
        slotB = (bufB, buf2B, buf3B, stB, gsemB, ssemB)

        def row_geom(r):
            ln = N - r
            o_r = r * N - ((r * (r - 1)) >> 1)
            t_r = (r * (r + 1)) >> 1
            a_beg = _ceil16(o_r)
            b_beg = a_beg - 16
            gs = ((b_beg + t_r) >> 3) << 3
            u = b_beg + t_r - gs
            t_p = t_r - r
            gs2 = ((b_beg + t_p) >> 3) << 3
            u2 = b_beg + t_p - gs2
            w16 = (((o_r + ln) >> 4) << 4) - a_beg
            return ln, o_r, a_beg, b_beg, gs, u, gs2, u2, w16

        def gather_descs(r, SZ, slot):
            buf, _, buf3, _, gsem, _ = slot
            _, _, _, _, gs, _, gs2, _, _ = row_geom(r)
            d1 = pltpu.make_async_copy(
                x_hbm.at[pl.ds(pl.multiple_of(gs, 8), SZ)],
                buf.at[pl.ds(0, SZ)], gsem)
            d2 = pltpu.make_async_copy(
                x_hbm.at[pl.ds(pl.multiple_of(gs2, 8), 32)], buf3, gsem)
            return d1, d2

        def row0_gather_descs(slot):
            buf, _, buf3, _, gsem, _ = slot
            d1 = pltpu.make_async_copy(x_hbm.at[pl.ds(0, N)],
                                       buf.at[pl.ds(16, N)], gsem)
            d2 = pltpu.make_async_copy(x_hbm.at[pl.ds(0, 32)], buf3, gsem)
            return d1, d2

        def fire_row(r, SZ, slot, first_band):
            if first_band:
                @pl.when(r > 0)
                def _():
                    for d in gather_descs(r, SZ, slot):
                        d.start()

                @pl.when(r == 0)
                def _():
                    for d in row0_gather_descs(slot):
                        d.start()
            else:
                for d in gather_descs(r, SZ, slot):
                    d.start()

        def scatter_descs(r, C, slot):
            _, buf2, _, st, _, ssem = slot
            _, _, a_beg, b_beg, _, _, _, _, w16 = row_geom(r)
            d_st = pltpu.make_async_copy(
                st, out_hbm.at[pl.ds(pl.multiple_of(b_beg, 8), 16)], ssem)
            toff = w16 - C
            d_tail = pltpu.make_async_copy(
                buf2.at[pl.ds(pl.multiple_of(toff, 8), C)],
                out_hbm.at[pl.ds(pl.multiple_of(a_beg + toff, 8), C)], ssem)
            return d_st, d_tail

        def chunk_desc(r, C, j, slot):
            _, buf2, _, _, _, ssem = slot
            _, _, a_beg, _, _, _, _, _, _ = row_geom(r)
            off = j * C
            return pltpu.make_async_copy(
                buf2.at[pl.ds(pl.multiple_of(off, 8), C)],
                out_hbm.at[pl.ds(pl.multiple_of(a_beg + off, 8), C)], ssem)

        def row0_scatter_desc(slot):
            _, buf2, _, _, _, ssem = slot
            return pltpu.make_async_copy(buf2.at[pl.ds(0, N)],
                                         out_hbm.at[pl.ds(0, N)], ssem)

        def nfull_of(r, C):
            _, _, _, _, _, _, _, _, w16 = row_geom(r)
            return w16 // C

        def scatter_descs_direct(r, C, slot):
            buf, _, _, st, _, ssem = slot
            _, _, a_beg, b_beg, _, _, _, _, w16 = row_geom(r)
            d_st = pltpu.make_async_copy(
                st, out_hbm.at[pl.ds(pl.multiple_of(b_beg, 8), 16)], ssem)
            toff = w16 - C
            d_tail = pltpu.make_async_copy(
                buf.at[pl.ds(pl.multiple_of(16 + toff, 8), C)],
                out_hbm.at[pl.ds(pl.multiple_of(a_beg + toff, 8), C)], ssem)
            return d_st, d_tail

        def chunk_desc_direct(r, C, j, slot):
            buf, _, _, _, _, ssem = slot
            _, _, a_beg, _, _, _, _, _, _ = row_geom(r)
            off = j * C
            return pltpu.make_async_copy(
                buf.at[pl.ds(pl.multiple_of(16 + off, 8), C)],
                out_hbm.at[pl.ds(pl.multiple_of(a_beg + off, 8), C)], ssem)

        def fire_scatters(r, C, slot, first_band, u):
            del u
            def general():
                d_st, d_tail = scatter_descs(r, C, slot)
                d_st.start()
                d_tail.start()

                def body(j, carry):
                    chunk_desc(r, C, j, slot).start()
                    return carry

                lax.fori_loop(jnp.int32(0), nfull_of(r, C), body,
                              jnp.int32(0))

            if first_band:
                @pl.when(r > 0)
                def _():
                    general()

                @pl.when(r == 0)
                def _():
                    row0_scatter_desc(slot).start()
            else:
                general()

        def drain_scatters(r, C, slot, first_band):
            def general():
                d_st, d_tail = scatter_descs(r, C, slot)
                d_st.wait()
                d_tail.wait()

                def body(j, carry):
                    chunk_desc(r, C, j, slot).wait()
                    return carry

                lax.fori_loop(jnp.int32(0), nfull_of(r, C), body, jnp.int32(0))

            if first_band:
                @pl.when(r > 0)
                def _():
                    general()

                @pl.when(r == 0)
                def _():
                    row0_scatter_desc(slot).wait()
            else:
                general()

        def consume_row(r, P, C, SZ, slot, first_band):
            buf, buf2, buf3, st, gsem, ssem = slot
            ln, o_r, a_beg, b_beg, gs, u, gs2, u2, w16 = row_geom(r)

            if first_band:
                @pl.when(r > 0)
                def _():
                    for d in gather_descs(r, SZ, slot):
                        d.wait()

                @pl.when(r == 0)
                def _():
                    for d in row0_gather_descs(slot):
                        d.wait()
                u_eff = jnp.where(r == 0, jnp.int32(0), u)
                a_eff = jnp.where(r == 0, jnp.int32(0), a_beg)
                w16_eff = (((o_r + ln) >> 4) << 4) - a_eff
            else:
                for d in gather_descs(r, SZ, slot):
                    d.wait()
                u_eff, w16_eff = u, w16

            @plsc.parallel_loop(jnp.int32(0), w16_eff >> 4, unroll=4)
            def _(k):
                v = buf[pl.ds(16 + u_eff + 16 * k, 16)]
                buf2[pl.ds(pl.multiple_of(16 * k, 8), 16)] = v

            if first_band:
                @pl.when(r > 0)
                def _():
                    va = buf3[pl.ds(u2, 16)]
                    vb = buf[pl.ds(u, 16)]
                    st[...] = jnp.where(lanes + b_beg < o_r, va, vb)
            else:
                va = buf3[pl.ds(u2, 16)]
                vb = buf[pl.ds(u, 16)]
                st[...] = jnp.where(lanes + b_beg < o_r, va, vb)

            fire_scatters(r, C, slot, first_band, u_eff)

        for bi, (P, lo, hi) in enumerate(bands):
            C = 3 * P // 8 if P >= 256 else max(16, P // 4)
            SZ = P + 64
            fb = (bi == 0)
            i_lo, i_hi = lo // _NW, hi // _NW
            npairs = (i_hi - i_lo) // 2

            fire_row(w + _NW * i_lo, SZ, slotA, fb)

            if npairs > 0:
                fire_row(w + _NW * (i_lo + 1), SZ, slotB, False)

                def pair_body(t, carry, P=P, C=C, SZ=SZ, fb=fb, i_lo=i_lo,
                              npairs=npairs):
                    rA = w + _NW * (i_lo + 2 * t)
                    rB = rA + _NW

                    @pl.when(t > 0)
                    def _():
                        drain_scatters(rA - 2 * _NW, C, slotA, fb)

                    consume_row(rA, P, C, SZ, slotA, fb)

                    @pl.when(t + 1 < npairs)
                    def _():
                        fire_row(rA + 2 * _NW, SZ, slotA, False)

                    @pl.when(t > 0)
                    def _():
                        drain_scatters(rB - 2 * _NW, C, slotB, False)

                    consume_row(rB, P, C, SZ, slotB, False)

                    @pl.when(t + 1 < npairs)
                    def _():
                        fire_row(rB + 2 * _NW, SZ, slotB, False)
                    return carry

                lax.fori_loop(jnp.int32(0), jnp.int32(npairs), pair_body,
                              jnp.int32(0))
                r_lastA = w + _NW * (i_lo + 2 * (npairs - 1))
                drain_scatters(r_lastA, C, slotA, fb)
                drain_scatters(r_lastA + _NW, C, slotB, False)
            else:
                r_only = w + _NW * i_lo
                consume_row(r_only, P, C, SZ, slotA, fb)
                drain_scatters(r_only, C, slotA, fb)

        @pl.when(w == 1)
        def _():
            cdescs = []
            for j in range(32):
                r = corner_lo + j
                src0 = r * (N + 1)
                gs_j = (src0 >> 3) << 3
                u_j = src0 - gs_j
                size_j = min(56, N * N - gs_j)
                d = pltpu.make_async_copy(
                    x_hbm.at[pl.ds(gs_j, size_j)],
                    csrc.at[pl.ds(64 * j + 16, size_j)], gsemA)
                d.start()
                cdescs.append((d, u_j))
            for d, _ in cdescs:
                d.wait()
            o_c = [((corner_lo + j) * N
                    - ((corner_lo + j) * (corner_lo + j - 1)) // 2)
                   for j in range(33)]
            for t in range(33):
                base = corner_base + 16 * t
                vec = None
                for j in range(32):
                    if o_c[j + 1] <= base or o_c[j] >= base + 16:
                        continue
                    start = 64 * j + 16 + cdescs[j][1] + (base - o_c[j])
                    a = csrc[pl.ds(start, 16)]
                    if vec is None:
                        vec = a
                    else:
                        vec = jnp.where(lanes + base >= o_c[j], a, vec)
                cout[pl.ds(16 * t, 16)] = vec
            pltpu.sync_copy(cout, out_hbm.at[pl.ds(corner_base, corner_elems)])

    return triu_kernel


def kernel(x):
    n = x.shape[-1]
    flat = x.reshape(-1)
    return _build(n)(flat)


# speedup vs baseline: 6.4252x; 1.0338x over previous
import functools

import jax
import jax.numpy as jnp
from jax import lax
from jax.experimental import pallas as pl
from jax.experimental.pallas import tpu as pltpu
from jax.experimental.pallas import tpu_sc as plsc

_NC = 2
_NS = 16
_NW = _NC * _NS


def _ceil16(v):
    return ((v + 15) >> 4) << 4


@functools.cache
def _build(N: int):
    total = N * (N + 1) // 2
    corner_lo = N - 32
    corner_elems = 528
    corner_base = total - corner_elems

    bands = []
    P = N
    while P >= 64:
        bands.append((P, N - P, N - P // 2))
        P //= 2

    mesh = plsc.VectorSubcoreMesh(core_axis_name="c", subcore_axis_name="s")

    @functools.partial(
        pl.kernel,
        mesh=mesh,
        out_type=jax.ShapeDtypeStruct((total,), jnp.float32),
        scratch_types=[
            pltpu.VMEM((N + 64, ), jnp.float32),
            pltpu.VMEM((N + 64, ), jnp.float32),
            pltpu.VMEM((N + 16, ), jnp.float32),
            pltpu.VMEM((N + 16, ), jnp.float32),
            pltpu.VMEM((32, ), jnp.float32),
            pltpu.VMEM((32, ), jnp.float32),
            pltpu.VMEM((16, ), jnp.float32),
            pltpu.VMEM((16, ), jnp.float32),
            pltpu.VMEM((2112, ), jnp.float32),
            pltpu.VMEM((corner_elems, ), jnp.float32),
            pltpu.SemaphoreType.DMA,
            pltpu.SemaphoreType.DMA,
            pltpu.SemaphoreType.DMA,
            pltpu.SemaphoreType.DMA,
        ],
    )
    def triu_kernel(x_hbm, out_hbm, bufA, bufB, buf2A, buf2B, buf3A, buf3B,
                    stA, stB, csrc, cout, gsemA, gsemB, ssemA, ssemB):
        c = lax.axis_index("c")
        s = lax.axis_index("s")
        w = c * _NS + s

        lanes = lax.iota(jnp.int32, 16)
        slotA = (bufA, buf2A, buf3A, stA, gsemA, ssemA)
        slotB = (bufB, buf2B, buf3B, stB, gsemB, ssemB)

        def row_geom(r):
            ln = N - r
            o_r = r * N - ((r * (r - 1)) >> 1)
            t_r = (r * (r + 1)) >> 1
            a_beg = _ceil16(o_r)
            b_beg = a_beg - 16
            gs = ((b_beg + t_r) >> 3) << 3
            u = b_beg + t_r - gs
            t_p = t_r - r
            gs2 = ((b_beg + t_p) >> 3) << 3
            u2 = b_beg + t_p - gs2
            w16 = (((o_r + ln) >> 4) << 4) - a_beg
            return ln, o_r, a_beg, b_beg, gs, u, gs2, u2, w16

        def gather_descs(r, SZ, slot):
            buf, _, buf3, _, gsem, _ = slot
            _, _, _, _, gs, _, gs2, _, _ = row_geom(r)
            d1 = pltpu.make_async_copy(
                x_hbm.at[pl.ds(pl.multiple_of(gs, 8), SZ)],
                buf.at[pl.ds(0, SZ)], gsem)
            d2 = pltpu.make_async_copy(
                x_hbm.at[pl.ds(pl.multiple_of(gs2, 8), 32)], buf3, gsem)
            return d1, d2

        def row0_gather_descs(slot):
            buf, _, buf3, _, gsem, _ = slot
            d1 = pltpu.make_async_copy(x_hbm.at[pl.ds(0, N)],
                                       buf.at[pl.ds(16, N)], gsem)
            d2 = pltpu.make_async_copy(x_hbm.at[pl.ds(0, 32)], buf3, gsem)
            return d1, d2

        def fire_row(r, SZ, slot, first_band):
            if first_band:
                @pl.when(r > 0)
                def _():
                    for d in gather_descs(r, SZ, slot):
                        d.start()

                @pl.when(r == 0)
                def _():
                    for d in row0_gather_descs(slot):
                        d.start()
            else:
                for d in gather_descs(r, SZ, slot):
                    d.start()

        def scatter_descs(r, C, slot):
            _, buf2, _, st, _, ssem = slot
            _, _, a_beg, b_beg, _, _, _, _, w16 = row_geom(r)
            d_st = pltpu.make_async_copy(
                st, out_hbm.at[pl.ds(pl.multiple_of(b_beg, 8), 16)], ssem)
            toff = w16 - C
            d_tail = pltpu.make_async_copy(
                buf2.at[pl.ds(pl.multiple_of(toff, 8), C)],
                out_hbm.at[pl.ds(pl.multiple_of(a_beg + toff, 8), C)], ssem)
            return d_st, d_tail

        def chunk_desc(r, C, j, slot):
            _, buf2, _, _, _, ssem = slot
            _, _, a_beg, _, _, _, _, _, _ = row_geom(r)
            off = j * C
            return pltpu.make_async_copy(
                buf2.at[pl.ds(pl.multiple_of(off, 8), C)],
                out_hbm.at[pl.ds(pl.multiple_of(a_beg + off, 8), C)], ssem)

        def row0_scatter_desc(slot):
            _, buf2, _, _, _, ssem = slot
            return pltpu.make_async_copy(buf2.at[pl.ds(0, N)],
                                         out_hbm.at[pl.ds(0, N)], ssem)

        def nfull_of(r, C):
            _, _, _, _, _, _, _, _, w16 = row_geom(r)
            return w16 // C

        def scatter_descs_direct(r, C, slot):
            buf, _, _, st, _, ssem = slot
            _, _, a_beg, b_beg, _, _, _, _, w16 = row_geom(r)
            d_st = pltpu.make_async_copy(
                st, out_hbm.at[pl.ds(pl.multiple_of(b_beg, 8), 16)], ssem)
            toff = w16 - C
            d_tail = pltpu.make_async_copy(
                buf.at[pl.ds(pl.multiple_of(16 + toff, 8), C)],
                out_hbm.at[pl.ds(pl.multiple_of(a_beg + toff, 8), C)], ssem)
            return d_st, d_tail

        def chunk_desc_direct(r, C, j, slot):
            buf, _, _, _, _, ssem = slot
            _, _, a_beg, _, _, _, _, _, _ = row_geom(r)
            off = j * C
            return pltpu.make_async_copy(
                buf.at[pl.ds(pl.multiple_of(16 + off, 8), C)],
                out_hbm.at[pl.ds(pl.multiple_of(a_beg + off, 8), C)], ssem)

        def fire_scatters(r, C, slot, first_band, u):
            del u
            def general():
                d_st, d_tail = scatter_descs(r, C, slot)
                d_st.start()
                d_tail.start()

                def body(j, carry):
                    chunk_desc(r, C, j, slot).start()
                    return carry

                lax.fori_loop(jnp.int32(0), nfull_of(r, C), body,
                              jnp.int32(0))

            if first_band:
                @pl.when(r > 0)
                def _():
                    general()

                @pl.when(r == 0)
                def _():
                    row0_scatter_desc(slot).start()
            else:
                general()

        def drain_scatters(r, C, slot, first_band):
            def general():
                d_st, d_tail = scatter_descs(r, C, slot)
                d_st.wait()
                d_tail.wait()

                def body(j, carry):
                    chunk_desc(r, C, j, slot).wait()
                    return carry

                lax.fori_loop(jnp.int32(0), nfull_of(r, C), body, jnp.int32(0))

            if first_band:
                @pl.when(r > 0)
                def _():
                    general()

                @pl.when(r == 0)
                def _():
                    row0_scatter_desc(slot).wait()
            else:
                general()

        def consume_row(r, P, C, SZ, slot, first_band):
            buf, buf2, buf3, st, gsem, ssem = slot
            ln, o_r, a_beg, b_beg, gs, u, gs2, u2, w16 = row_geom(r)

            if first_band:
                @pl.when(r > 0)
                def _():
                    for d in gather_descs(r, SZ, slot):
                        d.wait()

                @pl.when(r == 0)
                def _():
                    for d in row0_gather_descs(slot):
                        d.wait()
                u_eff = jnp.where(r == 0, jnp.int32(0), u)
                a_eff = jnp.where(r == 0, jnp.int32(0), a_beg)
                w16_eff = (((o_r + ln) >> 4) << 4) - a_eff
            else:
                for d in gather_descs(r, SZ, slot):
                    d.wait()
                u_eff, w16_eff = u, w16

            @plsc.parallel_loop(jnp.int32(0), w16_eff >> 4, unroll=8)
            def _(k):
                v = buf[pl.ds(16 + u_eff + 16 * k, 16)]
                buf2[pl.ds(pl.multiple_of(16 * k, 8), 16)] = v

            if first_band:
                @pl.when(r > 0)
                def _():
                    va = buf3[pl.ds(u2, 16)]
                    vb = buf[pl.ds(u, 16)]
                    st[...] = jnp.where(lanes + b_beg < o_r, va, vb)
            else:
                va = buf3[pl.ds(u2, 16)]
                vb = buf[pl.ds(u, 16)]
                st[...] = jnp.where(lanes + b_beg < o_r, va, vb)

            fire_scatters(r, C, slot, first_band, u_eff)

        for bi, (P, lo, hi) in enumerate(bands):
            C = 3 * P // 8 if P >= 256 else max(16, P // 4)
            SZ = P + 64
            fb = (bi == 0)
            i_lo, i_hi = lo // _NW, hi // _NW
            npairs = (i_hi - i_lo) // 2

            fire_row(w + _NW * i_lo, SZ, slotA, fb)

            if npairs > 0:
                fire_row(w + _NW * (i_lo + 1), SZ, slotB, False)

                def pair_body(t, carry, P=P, C=C, SZ=SZ, fb=fb, i_lo=i_lo,
                              npairs=npairs):
                    rA = w + _NW * (i_lo + 2 * t)
                    rB = rA + _NW

                    @pl.when(t > 0)
                    def _():
                        drain_scatters(rA - 2 * _NW, C, slotA, fb)

                    consume_row(rA, P, C, SZ, slotA, fb)

                    @pl.when(t + 1 < npairs)
                    def _():
                        fire_row(rA + 2 * _NW, SZ, slotA, False)

                    @pl.when(t > 0)
                    def _():
                        drain_scatters(rB - 2 * _NW, C, slotB, False)

                    consume_row(rB, P, C, SZ, slotB, False)

                    @pl.when(t + 1 < npairs)
                    def _():
                        fire_row(rB + 2 * _NW, SZ, slotB, False)
                    return carry

                lax.fori_loop(jnp.int32(0), jnp.int32(npairs), pair_body,
                              jnp.int32(0))
                r_lastA = w + _NW * (i_lo + 2 * (npairs - 1))
                drain_scatters(r_lastA, C, slotA, fb)
                drain_scatters(r_lastA + _NW, C, slotB, False)
            else:
                r_only = w + _NW * i_lo
                consume_row(r_only, P, C, SZ, slotA, fb)
                drain_scatters(r_only, C, slotA, fb)

        @pl.when(w == 1)
        def _():
            cdescs = []
            for j in range(32):
                r = corner_lo + j
                src0 = r * (N + 1)
                gs_j = (src0 >> 3) << 3
                u_j = src0 - gs_j
                size_j = min(56, N * N - gs_j)
                d = pltpu.make_async_copy(
                    x_hbm.at[pl.ds(gs_j, size_j)],
                    csrc.at[pl.ds(64 * j + 16, size_j)], gsemA)
                d.start()
                cdescs.append((d, u_j))
            for d, _ in cdescs:
                d.wait()
            o_c = [((corner_lo + j) * N
                    - ((corner_lo + j) * (corner_lo + j - 1)) // 2)
                   for j in range(33)]
            for t in range(33):
                base = corner_base + 16 * t
                vec = None
                for j in range(32):
                    if o_c[j + 1] <= base or o_c[j] >= base + 16:
                        continue
                    start = 64 * j + 16 + cdescs[j][1] + (base - o_c[j])
                    a = csrc[pl.ds(start, 16)]
                    if vec is None:
                        vec = a
                    else:
                        vec = jnp.where(lanes + base >= o_c[j], a, vec)
                cout[pl.ds(16 * t, 16)] = vec
            pltpu.sync_copy(cout, out_hbm.at[pl.ds(corner_base, corner_elems)])

    return triu_kernel


def kernel(x):
    n = x.shape[-1]
    flat = x.reshape(-1)
    return _build(n)(flat)
